# baseline (speedup 1.0000x reference)
import functools

import jax
import jax.numpy as jnp
from jax import lax
from jax.experimental import pallas as pl
from jax.experimental.pallas import tpu as pltpu
from jax.experimental.pallas import tpu_sc as plsc

B = 4096
L = 200
D = 64
NC = 2
NS = 16
NW = NC * NS
BPW = B // NW
CH = 100
NCH = L // CH
LANES = 16
NVR = D // LANES

_mesh = plsc.VectorSubcoreMesh(core_axis_name="c", subcore_axis_name="s")


@functools.partial(
    pl.kernel,
    mesh=_mesh,
    out_type=jax.ShapeDtypeStruct((B, D), jnp.float32),
    compiler_params=pltpu.CompilerParams(use_tc_tiling_on_sc=False),
    scratch_types=[
        pltpu.VMEM((BPW * NCH, CH), jnp.int32),
        pltpu.VMEM((2, NCH, CH, D), jnp.float32),
        pltpu.VMEM((BPW, D), jnp.float32),
        pltpu.SemaphoreType.DMA((2,)),
    ],
)
def _emb_pool(idx_hbm, tab_hbm, out_hbm, idx_v, gbuf, out_v, sem):
    wid = lax.axis_index("s") * NC + lax.axis_index("c")
    row0 = wid * (BPW * NCH)
    pltpu.sync_copy(idx_hbm.at[pl.ds(row0, BPW * NCH)], idx_v)

    def issue(s, b):
        for c in range(NCH):
            pltpu.make_async_copy(
                tab_hbm.at[idx_v.at[s * NCH + c]],
                gbuf.at[b, c],
                sem.at[b],
            ).start()

    def wait(b):
        for c in range(NCH):
            pltpu.make_async_copy(
                tab_hbm.at[idx_v.at[c]],
                gbuf.at[b, c],
                sem.at[b],
            ).wait()

    def accum_store(s, b):
        zero = jnp.zeros((LANES,), jnp.float32)
        acc = (zero,) * NVR

        def row(j, acc):
            return tuple(
                acc[k] + gbuf[b, c, j, pl.ds(k * LANES, LANES)]
                for k in range(NVR)
            )

        for c in range(NCH):
            def body4(j4, acc, c=c):
                for r in range(4):
                    acc = row(j4 * 4 + r, acc)
                return acc

            acc = lax.fori_loop(0, CH // 4, body4, acc)

        for k in range(NVR):
            out_v[s, pl.ds(k * LANES, LANES)] = acc[k]

    issue(0, 0)

    @pl.loop(0, BPW, step=2)
    def _(s):
        issue(s + 1, 1)
        wait(0)
        accum_store(s, 0)

        @pl.when(s + 2 < BPW)
        def _():
            issue(s + 2, 0)

        wait(1)
        accum_store(s + 1, 1)

    pltpu.sync_copy(out_v, out_hbm.at[pl.ds(wid * BPW, BPW)])


def kernel(indices, table):
    idx2 = indices.astype(jnp.int32).reshape(B * L // CH, CH)
    return _emb_pool(idx2, table)
